# Initial kernel scaffold; baseline (speedup 1.0000x reference)
import functools

import jax
import jax.numpy as jnp
from jax import lax
from jax.experimental import pallas as pl
from jax.experimental.pallas import tpu as pltpu
from jax.experimental.pallas import tpu_sc as plsc

N = 10000
E = 160000
IN_FEAT = 256
OUT_FEAT = 256
H = 4
R = 32

TC_BLK = 640
TC_GRID = E // TC_BLK

NC = 2
NS = 16
K = 80
EDGES_PER_TILE = E // NS
NCHUNK = EDGES_PER_TILE // K
HALF = N // NC
ACC_ROWS = 5120
ROWS_PER_TILE = ACC_ROWS // NS
DUMMY = 5100


def _tc_body(rel_ref, e_ref, w_ref, a_ref, out_ref):
    relv = rel_ref[0, 0, :]
    oh = (relv[:, None] == lax.broadcasted_iota(jnp.int32, (TC_BLK, R), 1))
    oh = oh.astype(jnp.float32)
    fe = jnp.dot(e_ref[...], w_ref[...], preferred_element_type=jnp.float32)
    w = jnp.dot(oh, a_ref[...], preferred_element_type=jnp.float32)
    t = w * fe
    t = jnp.where(t >= 0, t, 0.2 * t)
    out_ref[...] = (t[:, 0:OUT_FEAT] + t[:, OUT_FEAT:2 * OUT_FEAT]
                    + t[:, 2 * OUT_FEAT:3 * OUT_FEAT]
                    + t[:, 3 * OUT_FEAT:4 * OUT_FEAT])


def _edge_feat(e, W_fc, attn, rel):
    rel3 = rel.reshape(TC_GRID, 1, TC_BLK)
    attn2d = attn.reshape(R, H * OUT_FEAT)
    return pl.pallas_call(
        _tc_body,
        grid=(TC_GRID,),
        in_specs=[
            pl.BlockSpec((1, 1, TC_BLK), lambda i: (i, 0, 0)),
            pl.BlockSpec((TC_BLK, IN_FEAT), lambda i: (i, 0)),
            pl.BlockSpec((IN_FEAT, H * OUT_FEAT), lambda i: (0, 0)),
            pl.BlockSpec((R, H * OUT_FEAT), lambda i: (0, 0)),
        ],
        out_specs=pl.BlockSpec((TC_BLK, OUT_FEAT), lambda i: (i, 0)),
        out_shape=jax.ShapeDtypeStruct((E, OUT_FEAT), jnp.float32),
    )(rel3, e, W_fc, attn2d)


def _sc_body(src_hbm, dst_hbm, x_hbm, feat_hbm, zero_hbm, out_hbm,
             srcv, dstv, dstlocv, xrowsv, featv, acc, sem):
    c = lax.axis_index("c")
    s = lax.axis_index("s")

    pltpu.sync_copy(zero_hbm, featv)
    for k2 in range(ROWS_PER_TILE // K):
        off = pl.multiple_of(s * ROWS_PER_TILE + k2 * K, 8)
        pltpu.sync_copy(featv, acc.at[pl.ds(off, K)])
    plsc.subcore_barrier()

    def chunk(i, carry):
        base = pl.multiple_of(s * EDGES_PER_TILE + i * K, 8)
        pltpu.sync_copy(src_hbm.at[pl.ds(base, K)], srcv)
        pltpu.sync_copy(dst_hbm.at[pl.ds(base, K)], dstv)
        gather = pltpu.async_copy(x_hbm.at[srcv], xrowsv, sem)
        pltpu.sync_copy(feat_hbm.at[pl.ds(base, K)], featv)
        lo = c * HALF
        for j in range(K // 16):
            d = dstv[pl.ds(j * 16, 16)]
            loc = d - lo
            ok = (loc >= 0) & (loc < HALF)
            dstlocv[pl.ds(j * 16, 16)] = jnp.where(ok, loc, DUMMY)
        gather.wait()
        pltpu.sync_copy(featv, acc.at[dstlocv], add=True)
        pltpu.sync_copy(xrowsv, acc.at[dstlocv], add=True)
        return carry

    lax.fori_loop(0, NCHUNK, chunk, 0)
    plsc.subcore_barrier()

    for k2 in range(ROWS_PER_TILE // K):
        off = pl.multiple_of(s * ROWS_PER_TILE + k2 * K, 8)
        pltpu.sync_copy(acc.at[pl.ds(off, K)], xrowsv)
        pltpu.sync_copy(xrowsv, out_hbm.at[pl.ds(c * ACC_ROWS + off, K)])


_sc_scatter = functools.partial(
    pl.kernel,
    out_type=jax.ShapeDtypeStruct((NC * ACC_ROWS, OUT_FEAT), jnp.float32),
    mesh=plsc.VectorSubcoreMesh(core_axis_name="c", subcore_axis_name="s"),
    scratch_types=[
        pltpu.VMEM((K,), jnp.int32),
        pltpu.VMEM((K,), jnp.int32),
        pltpu.VMEM((K,), jnp.int32),
        pltpu.VMEM((K, OUT_FEAT), jnp.float32),
        pltpu.VMEM((K, OUT_FEAT), jnp.float32),
        pltpu.VMEM_SHARED((ACC_ROWS, OUT_FEAT), jnp.float32),
        pltpu.SemaphoreType.DMA,
    ],
)(_sc_body)


def kernel(x, e, W_fc, attn, edge_index, rel):
    src = edge_index[0].astype(jnp.int32)
    dst = edge_index[1].astype(jnp.int32)
    feat = _edge_feat(e, W_fc, attn, rel.astype(jnp.int32))
    zero = jnp.zeros((K, OUT_FEAT), jnp.float32)
    out = _sc_scatter(src, dst, x, feat, zero)
    return jnp.concatenate([out[:HALF], out[ACC_ROWS:ACC_ROWS + HALF]], axis=0)


# speedup vs baseline: 1.8299x; 1.8299x over previous
import functools

import jax
import jax.numpy as jnp
from jax import lax
from jax.experimental import pallas as pl
from jax.experimental.pallas import tpu as pltpu
from jax.experimental.pallas import tpu_sc as plsc

N = 10000
E = 160000
IN_FEAT = 256
OUT_FEAT = 256
H = 4
R = 32

TC_BLK = 640
TC_GRID = E // TC_BLK

NC = 2
NS = 16
NW = NC * NS
ROWS = 320
NPAD = NW * ROWS
GK = 64
CH = 512
NCHUNK = E // CH
REM = E - NCHUNK * CH
CAP = 96


def _tc_body(rel_ref, e_ref, w_ref, a_ref, out_ref):
    relv = rel_ref[0, 0, :]
    oh = (relv[:, None] == lax.broadcasted_iota(jnp.int32, (TC_BLK, R), 1))
    oh = oh.astype(jnp.float32)
    fe = jnp.dot(e_ref[...], w_ref[...], preferred_element_type=jnp.float32)
    w = jnp.dot(oh, a_ref[...], preferred_element_type=jnp.float32)
    t = w * fe
    t = jnp.where(t >= 0, t, 0.2 * t)
    out_ref[...] = (t[:, 0:OUT_FEAT] + t[:, OUT_FEAT:2 * OUT_FEAT]
                    + t[:, 2 * OUT_FEAT:3 * OUT_FEAT]
                    + t[:, 3 * OUT_FEAT:4 * OUT_FEAT])


def _edge_feat(e, W_fc, attn, rel):
    rel3 = rel.reshape(TC_GRID, 1, TC_BLK)
    attn2d = attn.reshape(R, H * OUT_FEAT)
    return pl.pallas_call(
        _tc_body,
        grid=(TC_GRID,),
        in_specs=[
            pl.BlockSpec((1, 1, TC_BLK), lambda i: (i, 0, 0)),
            pl.BlockSpec((TC_BLK, IN_FEAT), lambda i: (i, 0)),
            pl.BlockSpec((IN_FEAT, H * OUT_FEAT), lambda i: (0, 0)),
            pl.BlockSpec((R, H * OUT_FEAT), lambda i: (0, 0)),
        ],
        out_specs=pl.BlockSpec((TC_BLK, OUT_FEAT), lambda i: (i, 0)),
        out_shape=jax.ShapeDtypeStruct((E, OUT_FEAT), jnp.float32),
    )(rel3, e, W_fc, attn2d)


def _sc_body(src_hbm, dst_hbm, x_hbm, feat_hbm, out_hbm,
             dstb, srcb, locl, srcl, eidl, msg, featb, acc):
    c = lax.axis_index("c")
    s = lax.axis_index("s")
    w = c * NS + s
    lo = w * ROWS

    iota16 = lax.broadcasted_iota(jnp.int32, (16,), 0)
    zeros16 = jnp.zeros((16,), jnp.float32)

    def zrow(r, carry):
        for j in range(OUT_FEAT // 16):
            acc[pl.ds(r * OUT_FEAT + j * 16, 16)] = zeros16
        return carry
    lax.fori_loop(0, ROWS + 1, zrow, 0)

    def accumulate():
        def arow(r, carry):
            rv = jnp.zeros((16,), jnp.int32) + r
            locb = plsc.load_gather(locl, [rv]) * OUT_FEAT
            for j in range(OUT_FEAT // 16):
                idx = locb + (j * 16) + iota16
                a = plsc.load_gather(acc, [idx])
                cs = pl.ds(j * 16, 16)
                plsc.store_scatter(acc, [idx], a + msg[r, cs] + featb[r, cs])
            return carry
        lax.fori_loop(0, GK, arow, 0)

    def drain128():
        pltpu.sync_copy(x_hbm.at[srcl.at[pl.ds(0, GK)]], msg)
        pltpu.sync_copy(feat_hbm.at[eidl.at[pl.ds(0, GK)]], featb)
        accumulate()
        for l in (locl, srcl, eidl):
            g = l[pl.ds(GK, 16)]
            l[pl.ds(0, 16)] = g

    def compress_group(base, gi, off):
        d = dstb[pl.ds(gi * 16, 16)]
        sv = srcb[pl.ds(gi * 16, 16)]
        loc = d - lo
        m = (loc >= 0) & (loc < ROWS)
        pos = off + jnp.cumsum(jnp.where(m, 1, 0)) - 1
        plsc.store_scatter(locl, [pos], loc, mask=m)
        plsc.store_scatter(srcl, [pos], sv, mask=m)
        plsc.store_scatter(eidl, [pos], base + iota16, mask=m)
        return off + plsc.all_reduce_population_count(m)

    def scan_chunk(base, off, nedges):
        pltpu.sync_copy(dst_hbm.at[pl.ds(base, nedges)],
                        dstb if nedges == CH else dstb.at[pl.ds(0, nedges)])
        pltpu.sync_copy(src_hbm.at[pl.ds(base, nedges)],
                        srcb if nedges == CH else srcb.at[pl.ds(0, nedges)])

        def group(g, off):
            off = compress_group(base + g * 16, g, off)
            full = off >= GK
            do_drain = jnp.all(full)

            @pl.when(do_drain)
            def _():
                drain128()

            return jnp.where(full, off - GK, off)

        return lax.fori_loop(0, nedges // 16, group, off)

    def chunk(i, off):
        return scan_chunk(i * CH, off, CH)

    off0 = jnp.zeros((16,), jnp.int32)
    off = lax.fori_loop(0, NCHUNK, chunk, off0)
    off = scan_chunk(NCHUNK * CH, off, REM)

    for g in range(GK // 16):
        valid = (g * 16 + iota16) < off
        for l, pad in ((srcl, 0), (eidl, 0), (locl, ROWS)):
            v = l[pl.ds(g * 16, 16)]
            l[pl.ds(g * 16, 16)] = jnp.where(valid, v, pad)
    drain128()

    pltpu.sync_copy(acc.at[pl.ds(0, ROWS * OUT_FEAT)],
                    out_hbm.at[pl.ds(lo * OUT_FEAT, ROWS * OUT_FEAT)])


@functools.cache
def _sc_scatter():
    return pl.kernel(
        _sc_body,
        out_type=jax.ShapeDtypeStruct((NPAD * OUT_FEAT,), jnp.float32),
        mesh=plsc.VectorSubcoreMesh(core_axis_name="c", subcore_axis_name="s",
                                    num_cores=NC, num_subcores=NS),
        compiler_params=pltpu.CompilerParams(needs_layout_passes=False),
        scratch_types=[
            pltpu.VMEM((CH,), jnp.int32),
            pltpu.VMEM((CH,), jnp.int32),
            pltpu.VMEM((CAP,), jnp.int32),
            pltpu.VMEM((CAP,), jnp.int32),
            pltpu.VMEM((CAP,), jnp.int32),
            pltpu.VMEM((GK, OUT_FEAT), jnp.float32),
            pltpu.VMEM((GK, OUT_FEAT), jnp.float32),
            pltpu.VMEM(((ROWS + 1) * OUT_FEAT,), jnp.float32),
        ],
    )


def kernel(x, e, W_fc, attn, edge_index, rel):
    src = edge_index[0].astype(jnp.int32)
    dst = edge_index[1].astype(jnp.int32)
    feat = _edge_feat(e, W_fc, attn, rel.astype(jnp.int32))
    out = _sc_scatter()(src, dst, x, feat)
    return out.reshape(NPAD, OUT_FEAT)[:N]


# speedup vs baseline: 2.3345x; 1.2757x over previous
import functools

import jax
import jax.numpy as jnp
from jax import lax
from jax.experimental import pallas as pl
from jax.experimental.pallas import tpu as pltpu
from jax.experimental.pallas import tpu_sc as plsc

N = 10000
E = 160000
IN_FEAT = 256
OUT_FEAT = 256
H = 4
R = 32

TC_BLK = 640
TC_GRID = E // TC_BLK

NC = 2
NS = 16
NW = NC * NS
ROWS = 320
NPAD = NW * ROWS
GK = 64
CH = 2048
NCHUNK = E // CH
REM = E - NCHUNK * CH
CAP = 96


def _tc_body(rel_ref, e_ref, w_ref, a_ref, out_ref):
    relv = rel_ref[0, 0, :]
    oh = (relv[:, None] == lax.broadcasted_iota(jnp.int32, (TC_BLK, R), 1))
    oh = oh.astype(jnp.float32)
    fe = jnp.dot(e_ref[...], w_ref[...], preferred_element_type=jnp.float32)
    w = jnp.dot(oh, a_ref[...], preferred_element_type=jnp.float32)
    t = w * fe
    t = jnp.where(t >= 0, t, 0.2 * t)
    out_ref[...] = (t[:, 0:OUT_FEAT] + t[:, OUT_FEAT:2 * OUT_FEAT]
                    + t[:, 2 * OUT_FEAT:3 * OUT_FEAT]
                    + t[:, 3 * OUT_FEAT:4 * OUT_FEAT])


def _edge_feat(e, W_fc, attn, rel):
    rel3 = rel.reshape(TC_GRID, 1, TC_BLK)
    attn2d = attn.reshape(R, H * OUT_FEAT)
    return pl.pallas_call(
        _tc_body,
        grid=(TC_GRID,),
        in_specs=[
            pl.BlockSpec((1, 1, TC_BLK), lambda i: (i, 0, 0)),
            pl.BlockSpec((TC_BLK, IN_FEAT), lambda i: (i, 0)),
            pl.BlockSpec((IN_FEAT, H * OUT_FEAT), lambda i: (0, 0)),
            pl.BlockSpec((R, H * OUT_FEAT), lambda i: (0, 0)),
        ],
        out_specs=pl.BlockSpec((TC_BLK, OUT_FEAT), lambda i: (i, 0)),
        out_shape=jax.ShapeDtypeStruct((E, OUT_FEAT), jnp.float32),
    )(rel3, e, W_fc, attn2d)


def _sc_body(src_hbm, dst_hbm, x_hbm, feat_hbm, out_hbm,
             dstb, srcb, dstb1, srcb1, locl, srcl, eidl, msg, featb, acc,
             semA, semB, g1, g2):
    c = lax.axis_index("c")
    s = lax.axis_index("s")
    w = c * NS + s
    lo = w * ROWS

    iota16 = lax.broadcasted_iota(jnp.int32, (16,), 0)
    zeros16 = jnp.zeros((16,), jnp.float32)

    def zrow(r, carry):
        for j in range(OUT_FEAT // 16):
            acc[pl.ds(r * OUT_FEAT + j * 16, 16)] = zeros16
        return carry
    lax.fori_loop(0, ROWS + 1, zrow, 0)

    def accumulate():
        def arow(r, carry):
            rv = jnp.zeros((16,), jnp.int32) + r
            locb = plsc.load_gather(locl, [rv]) * OUT_FEAT
            for j in range(OUT_FEAT // 16):
                idx = locb + (j * 16) + iota16
                a = plsc.load_gather(acc, [idx])
                cs = pl.ds(j * 16, 16)
                plsc.store_scatter(acc, [idx], a + msg[r, cs] + featb[r, cs])
            return carry
        lax.fori_loop(0, GK, arow, 0)

    def drain128():
        cp1 = pltpu.async_copy(x_hbm.at[srcl.at[pl.ds(0, GK)]], msg, g1)
        cp2 = pltpu.async_copy(feat_hbm.at[eidl.at[pl.ds(0, GK)]], featb, g2)
        cp1.wait()
        cp2.wait()
        accumulate()
        for l in (locl, srcl, eidl):
            g = l[pl.ds(GK, 16)]
            l[pl.ds(0, 16)] = g

    def compress_group(dstb, srcb, base, gi, off):
        d = dstb[pl.ds(gi * 16, 16)]
        sv = srcb[pl.ds(gi * 16, 16)]
        loc = d - lo
        m = (loc >= 0) & (loc < ROWS)
        pos = off + jnp.cumsum(jnp.where(m, 1, 0)) - 1
        plsc.store_scatter(locl, [pos], loc, mask=m)
        plsc.store_scatter(srcl, [pos], sv, mask=m)
        plsc.store_scatter(eidl, [pos], base + iota16, mask=m)
        return off + plsc.all_reduce_population_count(m)

    def scan_groups(dstb, srcb, base, off, nedges):
        def group(g, off):
            off = compress_group(dstb, srcb, base + g * 16, g, off)
            full = off >= GK
            do_drain = jnp.all(full)

            @pl.when(do_drain)
            def _():
                drain128()

            return jnp.where(full, off - GK, off)

        return lax.fori_loop(0, nedges // 16, group, off)

    def start_load(ci, dbuf, sbuf, sem):
        base = ci * CH
        pltpu.async_copy(dst_hbm.at[pl.ds(base, CH)], dbuf, sem)
        pltpu.async_copy(src_hbm.at[pl.ds(base, CH)], sbuf, sem)

    def wait_load(dbuf, sbuf, sem):
        pltpu.make_async_copy(dst_hbm.at[pl.ds(0, CH)], dbuf, sem).wait()
        pltpu.make_async_copy(src_hbm.at[pl.ds(0, CH)], sbuf, sem).wait()

    NPAIR = NCHUNK // 2
    start_load(0, dstb, srcb, semA)
    start_load(1, dstb1, srcb1, semB)

    def pair(k, off):
        wait_load(dstb, srcb, semA)
        off = scan_groups(dstb, srcb, (2 * k) * CH, off, CH)

        @pl.when(k < NPAIR - 1)
        def _():
            start_load(2 * k + 2, dstb, srcb, semA)

        wait_load(dstb1, srcb1, semB)
        off = scan_groups(dstb1, srcb1, (2 * k + 1) * CH, off, CH)

        @pl.when(k < NPAIR - 1)
        def _():
            start_load(2 * k + 3, dstb1, srcb1, semB)

        return off

    off0 = jnp.zeros((16,), jnp.int32)
    off = lax.fori_loop(0, NPAIR, pair, off0)

    pltpu.sync_copy(dst_hbm.at[pl.ds(NCHUNK * CH, REM)], dstb.at[pl.ds(0, REM)])
    pltpu.sync_copy(src_hbm.at[pl.ds(NCHUNK * CH, REM)], srcb.at[pl.ds(0, REM)])
    off = scan_groups(dstb, srcb, NCHUNK * CH, off, REM)

    for g in range(GK // 16):
        valid = (g * 16 + iota16) < off
        for l, pad in ((srcl, 0), (eidl, 0), (locl, ROWS)):
            v = l[pl.ds(g * 16, 16)]
            l[pl.ds(g * 16, 16)] = jnp.where(valid, v, pad)
    drain128()

    pltpu.sync_copy(acc.at[pl.ds(0, ROWS * OUT_FEAT)],
                    out_hbm.at[pl.ds(lo * OUT_FEAT, ROWS * OUT_FEAT)])


@functools.cache
def _sc_scatter():
    return pl.kernel(
        _sc_body,
        out_type=jax.ShapeDtypeStruct((NPAD * OUT_FEAT,), jnp.float32),
        mesh=plsc.VectorSubcoreMesh(core_axis_name="c", subcore_axis_name="s",
                                    num_cores=NC, num_subcores=NS),
        compiler_params=pltpu.CompilerParams(needs_layout_passes=False),
        scratch_types=[
            pltpu.VMEM((CH,), jnp.int32),
            pltpu.VMEM((CH,), jnp.int32),
            pltpu.VMEM((CH,), jnp.int32),
            pltpu.VMEM((CH,), jnp.int32),
            pltpu.VMEM((CAP,), jnp.int32),
            pltpu.VMEM((CAP,), jnp.int32),
            pltpu.VMEM((CAP,), jnp.int32),
            pltpu.VMEM((GK, OUT_FEAT), jnp.float32),
            pltpu.VMEM((GK, OUT_FEAT), jnp.float32),
            pltpu.VMEM(((ROWS + 1) * OUT_FEAT,), jnp.float32),
            pltpu.SemaphoreType.DMA,
            pltpu.SemaphoreType.DMA,
            pltpu.SemaphoreType.DMA,
            pltpu.SemaphoreType.DMA,
        ],
    )


def kernel(x, e, W_fc, attn, edge_index, rel):
    src = edge_index[0].astype(jnp.int32)
    dst = edge_index[1].astype(jnp.int32)
    feat = _edge_feat(e, W_fc, attn, rel.astype(jnp.int32))
    out = _sc_scatter()(src, dst, x, feat)
    return out.reshape(NPAD, OUT_FEAT)[:N]


# speedup vs baseline: 2.3361x; 1.0007x over previous
import functools

import jax
import jax.numpy as jnp
from jax import lax
from jax.experimental import pallas as pl
from jax.experimental.pallas import tpu as pltpu
from jax.experimental.pallas import tpu_sc as plsc

N = 10000
E = 160000
IN_FEAT = 256
OUT_FEAT = 256
H = 4
R = 32

TC_BLK = 1280
TC_GRID = E // TC_BLK

NC = 2
NS = 16
NW = NC * NS
ROWS = 320
NPAD = NW * ROWS
GK = 64
CH = 2048
NCHUNK = E // CH
REM = E - NCHUNK * CH
CAP = 96


def _tc_body(rel_ref, e_ref, w_ref, a_ref, out_ref):
    relv = rel_ref[0, 0, :]
    oh = (relv[:, None] == lax.broadcasted_iota(jnp.int32, (TC_BLK, R), 1))
    oh = oh.astype(jnp.bfloat16)
    fe = jnp.dot(e_ref[...], w_ref[...], preferred_element_type=jnp.float32)
    w = jnp.dot(oh, a_ref[...], preferred_element_type=jnp.float32)
    t = w * fe
    t = jnp.where(t >= 0, t, 0.2 * t)
    out_ref[...] = (t[:, 0:OUT_FEAT] + t[:, OUT_FEAT:2 * OUT_FEAT]
                    + t[:, 2 * OUT_FEAT:3 * OUT_FEAT]
                    + t[:, 3 * OUT_FEAT:4 * OUT_FEAT])


def _edge_feat(e, W_fc, attn, rel):
    rel3 = rel.reshape(TC_GRID, 1, TC_BLK)
    attn2d = attn.reshape(R, H * OUT_FEAT)
    return pl.pallas_call(
        _tc_body,
        grid=(TC_GRID,),
        in_specs=[
            pl.BlockSpec((1, 1, TC_BLK), lambda i: (i, 0, 0)),
            pl.BlockSpec((TC_BLK, IN_FEAT), lambda i: (i, 0)),
            pl.BlockSpec((IN_FEAT, H * OUT_FEAT), lambda i: (0, 0)),
            pl.BlockSpec((R, H * OUT_FEAT), lambda i: (0, 0)),
        ],
        out_specs=pl.BlockSpec((TC_BLK, OUT_FEAT), lambda i: (i, 0)),
        out_shape=jax.ShapeDtypeStruct((E, OUT_FEAT), jnp.float32),
    )(rel3, e, W_fc, attn2d)


def _sc_body(src_hbm, dst_hbm, x_hbm, feat_hbm, out_hbm,
             dstb, srcb, dstb1, srcb1, locl, srcl, eidl, msg, featb, acc,
             semA, semB, g1, g2):
    c = lax.axis_index("c")
    s = lax.axis_index("s")
    w = c * NS + s
    lo = w * ROWS

    iota16 = lax.broadcasted_iota(jnp.int32, (16,), 0)
    zeros16 = jnp.zeros((16,), jnp.float32)

    def zrow(r, carry):
        for j in range(OUT_FEAT // 16):
            acc[pl.ds(r * OUT_FEAT + j * 16, 16)] = zeros16
        return carry
    lax.fori_loop(0, ROWS + 1, zrow, 0)

    def accumulate():
        def arow(r, carry):
            rv = jnp.zeros((16,), jnp.int32) + r
            locb = plsc.load_gather(locl, [rv]) * OUT_FEAT
            for j in range(OUT_FEAT // 16):
                idx = locb + (j * 16) + iota16
                a = plsc.load_gather(acc, [idx])
                cs = pl.ds(j * 16, 16)
                plsc.store_scatter(acc, [idx], a + msg[r, cs] + featb[r, cs])
            return carry
        lax.fori_loop(0, GK, arow, 0)

    def drain128():
        cp1 = pltpu.async_copy(x_hbm.at[srcl.at[pl.ds(0, GK)]], msg, g1)
        cp2 = pltpu.async_copy(feat_hbm.at[eidl.at[pl.ds(0, GK)]], featb, g2)
        cp1.wait()
        cp2.wait()
        accumulate()
        for l in (locl, srcl, eidl):
            g = l[pl.ds(GK, 16)]
            l[pl.ds(0, 16)] = g

    def compress_group(dstb, srcb, base, gi, off):
        d = dstb[pl.ds(gi * 16, 16)]
        sv = srcb[pl.ds(gi * 16, 16)]
        loc = d - lo
        m = (loc >= 0) & (loc < ROWS)
        pos = off + jnp.cumsum(jnp.where(m, 1, 0)) - 1
        plsc.store_scatter(locl, [pos], loc, mask=m)
        plsc.store_scatter(srcl, [pos], sv, mask=m)
        plsc.store_scatter(eidl, [pos], base + iota16, mask=m)
        return off + plsc.all_reduce_population_count(m)

    def scan_groups(dstb, srcb, base, off, nedges):
        def group(g, off):
            off = compress_group(dstb, srcb, base + g * 16, g, off)
            full = off >= GK
            do_drain = jnp.all(full)

            @pl.when(do_drain)
            def _():
                drain128()

            return jnp.where(full, off - GK, off)

        return lax.fori_loop(0, nedges // 16, group, off)

    def start_load(ci, dbuf, sbuf, sem):
        base = ci * CH
        pltpu.async_copy(dst_hbm.at[pl.ds(base, CH)], dbuf, sem)
        pltpu.async_copy(src_hbm.at[pl.ds(base, CH)], sbuf, sem)

    def wait_load(dbuf, sbuf, sem):
        pltpu.make_async_copy(dst_hbm.at[pl.ds(0, CH)], dbuf, sem).wait()
        pltpu.make_async_copy(src_hbm.at[pl.ds(0, CH)], sbuf, sem).wait()

    NPAIR = NCHUNK // 2
    start_load(0, dstb, srcb, semA)
    start_load(1, dstb1, srcb1, semB)

    def pair(k, off):
        wait_load(dstb, srcb, semA)
        off = scan_groups(dstb, srcb, (2 * k) * CH, off, CH)

        @pl.when(k < NPAIR - 1)
        def _():
            start_load(2 * k + 2, dstb, srcb, semA)

        wait_load(dstb1, srcb1, semB)
        off = scan_groups(dstb1, srcb1, (2 * k + 1) * CH, off, CH)

        @pl.when(k < NPAIR - 1)
        def _():
            start_load(2 * k + 3, dstb1, srcb1, semB)

        return off

    off0 = jnp.zeros((16,), jnp.int32)
    off = lax.fori_loop(0, NPAIR, pair, off0)

    pltpu.sync_copy(dst_hbm.at[pl.ds(NCHUNK * CH, REM)], dstb.at[pl.ds(0, REM)])
    pltpu.sync_copy(src_hbm.at[pl.ds(NCHUNK * CH, REM)], srcb.at[pl.ds(0, REM)])
    off = scan_groups(dstb, srcb, NCHUNK * CH, off, REM)

    for g in range(GK // 16):
        valid = (g * 16 + iota16) < off
        for l, pad in ((srcl, 0), (eidl, 0), (locl, ROWS)):
            v = l[pl.ds(g * 16, 16)]
            l[pl.ds(g * 16, 16)] = jnp.where(valid, v, pad)
    drain128()

    pltpu.sync_copy(acc.at[pl.ds(0, ROWS * OUT_FEAT)],
                    out_hbm.at[pl.ds(lo * OUT_FEAT, ROWS * OUT_FEAT)])


@functools.cache
def _sc_scatter():
    return pl.kernel(
        _sc_body,
        out_type=jax.ShapeDtypeStruct((NPAD * OUT_FEAT,), jnp.float32),
        mesh=plsc.VectorSubcoreMesh(core_axis_name="c", subcore_axis_name="s",
                                    num_cores=NC, num_subcores=NS),
        compiler_params=pltpu.CompilerParams(needs_layout_passes=False),
        scratch_types=[
            pltpu.VMEM((CH,), jnp.int32),
            pltpu.VMEM((CH,), jnp.int32),
            pltpu.VMEM((CH,), jnp.int32),
            pltpu.VMEM((CH,), jnp.int32),
            pltpu.VMEM((CAP,), jnp.int32),
            pltpu.VMEM((CAP,), jnp.int32),
            pltpu.VMEM((CAP,), jnp.int32),
            pltpu.VMEM((GK, OUT_FEAT), jnp.float32),
            pltpu.VMEM((GK, OUT_FEAT), jnp.float32),
            pltpu.VMEM(((ROWS + 1) * OUT_FEAT,), jnp.float32),
            pltpu.SemaphoreType.DMA,
            pltpu.SemaphoreType.DMA,
            pltpu.SemaphoreType.DMA,
            pltpu.SemaphoreType.DMA,
        ],
    )


def kernel(x, e, W_fc, attn, edge_index, rel):
    src = edge_index[0].astype(jnp.int32)
    dst = edge_index[1].astype(jnp.int32)
    feat = _edge_feat(e.astype(jnp.bfloat16), W_fc.astype(jnp.bfloat16),
                      attn.astype(jnp.bfloat16), rel.astype(jnp.int32))
    out = _sc_scatter()(src, dst, x, feat)
    return out.reshape(NPAD, OUT_FEAT)[:N]
